# baseline (speedup 1.0000x reference)
import functools

import jax
import jax.numpy as jnp
import numpy as np
from jax import lax
from jax.experimental import pallas as pl
from jax.experimental.pallas import tpu as pltpu
from jax.experimental.pallas import tpu_sc as plsc

D_MODEL = 2048
N_EXP = 16
TOP_K = 2
N_TOK = 16384
TB = 2048

with jax.default_device(jax.devices("cpu")[0]):
    _EPS = np.asarray(
        jax.random.normal(jax.random.key(42), (N_TOK, N_EXP), dtype=jnp.float32)
    )

_NC, _NS, _L = 2, 16, 16
_NW = _NC * _NS
_TOKW = N_TOK // _NW
_NG = _TOKW // _L


def _tc_block(h_ref, wl_ref, wn_ref, bl_ref, bn_ref, eps_ref,
              noisy_ref, full_ref):
    dn = (((1,), (1,)), ((), ()))
    h = h_ref[...]
    logits = lax.dot_general(h, wl_ref[...], dn,
                             preferred_element_type=jnp.float32) + bl_ref[...]
    zn = lax.dot_general(h, wn_ref[...], dn,
                         preferred_element_type=jnp.float32) + bn_ref[...]
    noisy = logits + eps_ref[...] * jax.nn.softplus(zn)
    noisy_ref[...] = noisy
    m1 = jnp.max(noisy, axis=1, keepdims=True)
    e = jnp.exp(noisy - m1)
    full_ref[...] = e / jnp.sum(e, axis=1, keepdims=True)


def _sc_topk(noisy_hbm, route_hbm, ix_hbm, noisy_v, route_v, ix_v):
    wid = lax.axis_index("s") * _NC + lax.axis_index("c")
    pltpu.sync_copy(noisy_hbm.at[pl.ds(wid * _TOKW * N_EXP, _TOKW * N_EXP)],
                    noisy_v)
    lanes = lax.iota(jnp.int32, _L)
    _GRP = _L // TOP_K

    def group(g, carry):
        ixvec = jnp.zeros((_L,), jnp.int32)
        for k in range(_GRP):
            t = g * _GRP + k
            row = noisy_v[pl.ds(t * N_EXP, N_EXP)]
            m1 = jnp.max(row)
            a1 = jnp.min(jnp.where(row == m1, lanes, N_EXP))
            masked = jnp.where(lanes == a1, -jnp.inf, row)
            m2 = jnp.max(masked)
            a2 = jnp.min(jnp.where(masked == m2, lanes, N_EXP))
            e2 = jnp.exp(jnp.broadcast_to(m2 - m1, (_L,)))
            denom = 1.0 + e2
            p1 = 1.0 / denom
            p2 = e2 / denom
            zero = jnp.zeros((_L,), jnp.float32)
            route_v[pl.ds(t * N_EXP, N_EXP)] = jnp.where(
                lanes == a1, p1, jnp.where(lanes == a2, p2, zero))
            ixvec = jnp.where(lanes == TOP_K * k, a1, ixvec)
            ixvec = jnp.where(lanes == TOP_K * k + 1, a2, ixvec)
        ix_v[pl.ds(g * _L, _L)] = ixvec
        return carry

    lax.fori_loop(0, _TOKW // _GRP, group, 0)
    pltpu.sync_copy(route_v,
                    route_hbm.at[pl.ds(wid * _TOKW * N_EXP, _TOKW * N_EXP)])
    pltpu.sync_copy(ix_v, ix_hbm.at[pl.ds(wid * _TOKW * TOP_K, _TOKW * TOP_K)])


@functools.lru_cache(maxsize=1)
def _sc_topk_call():
    return pl.kernel(
        _sc_topk,
        mesh=plsc.VectorSubcoreMesh(core_axis_name="c", subcore_axis_name="s"),
        compiler_params=pltpu.CompilerParams(needs_layout_passes=False),
        out_type=[
            jax.ShapeDtypeStruct((N_TOK * N_EXP,), jnp.float32),
            jax.ShapeDtypeStruct((N_TOK * TOP_K,), jnp.int32),
        ],
        scratch_types=[
            pltpu.VMEM((_TOKW * N_EXP,), jnp.float32),
            pltpu.VMEM((_TOKW * N_EXP,), jnp.float32),
            pltpu.VMEM((_TOKW * TOP_K,), jnp.int32),
        ],
    )


def kernel(h, Wl, bl, Wn, bn):
    bl2 = bl.reshape(1, N_EXP)
    bn2 = bn.reshape(1, N_EXP)
    eps = jnp.asarray(_EPS)

    grid = (N_TOK // TB,)
    noisy, full_p = pl.pallas_call(
        _tc_block,
        grid=grid,
        in_specs=[
            pl.BlockSpec((TB, D_MODEL), lambda i: (i, 0)),
            pl.BlockSpec((N_EXP, D_MODEL), lambda i: (0, 0)),
            pl.BlockSpec((N_EXP, D_MODEL), lambda i: (0, 0)),
            pl.BlockSpec((1, N_EXP), lambda i: (0, 0)),
            pl.BlockSpec((1, N_EXP), lambda i: (0, 0)),
            pl.BlockSpec((TB, N_EXP), lambda i: (i, 0)),
        ],
        out_specs=[
            pl.BlockSpec((TB, N_EXP), lambda i: (i, 0)),
            pl.BlockSpec((TB, N_EXP), lambda i: (i, 0)),
        ],
        out_shape=[
            jax.ShapeDtypeStruct((N_TOK, N_EXP), jnp.float32),
            jax.ShapeDtypeStruct((N_TOK, N_EXP), jnp.float32),
        ],
    )(h, Wl, Wn, bl2, bn2, eps)

    route_flat, ix_flat = _sc_topk_call()(noisy.reshape(-1))
    return (route_flat.reshape(N_TOK, N_EXP),
            ix_flat.reshape(N_TOK, TOP_K), full_p)


# speedup vs baseline: 1.4643x; 1.4643x over previous
import jax
import jax.numpy as jnp
import numpy as np
from jax.experimental import pallas as pl

D_MODEL = 2048
N_EXP = 16
TOP_K = 2
N_TOK = 16384
TB = 2048

with jax.default_device(jax.devices("cpu")[0]):
    _EPS = np.asarray(
        jax.random.normal(jax.random.key(42), (N_TOK, N_EXP), dtype=jnp.float32)
    )


def _router_block(h_ref, wl_ref, wn_ref, bl_ref, bn_ref, eps_ref,
                  route_ref, ix_ref, full_ref):
    dn = (((1,), (1,)), ((), ()))
    h = h_ref[...]
    logits = jax.lax.dot_general(h, wl_ref[...], dn,
                                 preferred_element_type=jnp.float32) + bl_ref[...]
    zn = jax.lax.dot_general(h, wn_ref[...], dn,
                             preferred_element_type=jnp.float32) + bn_ref[...]
    noisy = logits + eps_ref[...] * jax.nn.softplus(zn)

    m1 = jnp.max(noisy, axis=1, keepdims=True)
    e = jnp.exp(noisy - m1)
    full_ref[...] = e / jnp.sum(e, axis=1, keepdims=True)

    col = jax.lax.broadcasted_iota(jnp.int32, (TB, N_EXP), 1)
    a1 = jnp.argmax(noisy, axis=1, keepdims=True)
    masked = jnp.where(col == a1, -jnp.inf, noisy)
    v2 = jnp.max(masked, axis=1, keepdims=True)
    a2 = jnp.argmax(masked, axis=1, keepdims=True)
    ix_ref[...] = jnp.concatenate([a1, a2], axis=1)

    e2 = jnp.exp(v2 - m1)
    p1 = 1.0 / (1.0 + e2)
    p2 = e2 / (1.0 + e2)
    route_ref[...] = jnp.where(col == a1, p1, jnp.where(col == a2, p2, 0.0))


def kernel(h, Wl, bl, Wn, bn):
    bl2 = bl.reshape(1, N_EXP)
    bn2 = bn.reshape(1, N_EXP)
    eps = jnp.asarray(_EPS)

    grid = (N_TOK // TB,)
    route_p, ix, full_p = pl.pallas_call(
        _router_block,
        grid=grid,
        in_specs=[
            pl.BlockSpec((TB, D_MODEL), lambda i: (i, 0)),
            pl.BlockSpec((N_EXP, D_MODEL), lambda i: (0, 0)),
            pl.BlockSpec((N_EXP, D_MODEL), lambda i: (0, 0)),
            pl.BlockSpec((1, N_EXP), lambda i: (0, 0)),
            pl.BlockSpec((1, N_EXP), lambda i: (0, 0)),
            pl.BlockSpec((TB, N_EXP), lambda i: (i, 0)),
        ],
        out_specs=[
            pl.BlockSpec((TB, N_EXP), lambda i: (i, 0)),
            pl.BlockSpec((TB, TOP_K), lambda i: (i, 0)),
            pl.BlockSpec((TB, N_EXP), lambda i: (i, 0)),
        ],
        out_shape=[
            jax.ShapeDtypeStruct((N_TOK, N_EXP), jnp.float32),
            jax.ShapeDtypeStruct((N_TOK, TOP_K), jnp.int32),
            jax.ShapeDtypeStruct((N_TOK, N_EXP), jnp.float32),
        ],
    )(h, Wl, Wn, bl2, bn2, eps)
    return route_p, ix, full_p
